# baseline (speedup 1.0000x reference)
import jax
import jax.numpy as jnp
from jax import lax
from jax.experimental import pallas as pl
from jax.experimental.pallas import tpu as pltpu
from jax.experimental.pallas import tpu_sc as plsc

LATENT_DIM = 32
BATCH = 16384
NUM_CORES = 2
NUM_SUBCORES = 16
NUM_WORKERS = NUM_CORES * NUM_SUBCORES
B_PER_W = BATCH // NUM_WORKERS
NB = 4
LANE_COLS = 128
PAD_COLS = 132


def _pmf_body(uidx_hbm, iidx_hbm, wut_hbm, wit_hbm, out_hbm,
              uidx_v, iidx_v, ubufs, ibufs, out_v, usems, isems):
    wid = lax.axis_index("s") * NUM_CORES + lax.axis_index("c")
    base = wid * B_PER_W

    pltpu.sync_copy(uidx_hbm.at[pl.ds(base, B_PER_W)], uidx_v)
    pltpu.sync_copy(iidx_hbm.at[pl.ds(base, B_PER_W)], iidx_v)

    lane = lax.iota(jnp.int32, 16)

    def scalar_at(ref, chunk_base, off):
        return ref[pl.ds(chunk_base, 16)][off]

    def fire(iu, ii, slot):
        ucol0 = pl.multiple_of((iu >> 7) * LANE_COLS, LANE_COLS)
        icol0 = pl.multiple_of((ii >> 7) * LANE_COLS, LANE_COLS)
        pltpu.async_copy(wut_hbm.at[:, pl.ds(ucol0, LANE_COLS)],
                         ubufs.at[slot, :, pl.ds(0, LANE_COLS)],
                         usems.at[slot])
        pltpu.async_copy(wit_hbm.at[:, pl.ds(icol0, LANE_COLS)],
                         ibufs.at[slot, :, pl.ds(0, LANE_COLS)],
                         isems.at[slot])

    for s in range(NB):
        fire(scalar_at(uidx_v, 0, s), scalar_at(iidx_v, 0, s), s)

    def g_body(g, _):
        b0 = g * 16
        accv = jnp.zeros((16,), jnp.float32)
        for q in range(16 // NB):
            for s in range(NB):
                b = b0 + q * NB + s
                pltpu.make_async_copy(
                    wut_hbm.at[:, pl.ds(0, LANE_COLS)],
                    ubufs.at[s, :, pl.ds(0, LANE_COLS)], usems.at[s]).wait()
                pltpu.make_async_copy(
                    wit_hbm.at[:, pl.ds(0, LANE_COLS)],
                    ibufs.at[s, :, pl.ds(0, LANE_COLS)], isems.at[s]).wait()
                x = q * NB + s
                ucol = jnp.full((16,),
                                scalar_at(uidx_v, b0, x) & (LANE_COLS - 1),
                                jnp.int32)
                icol = jnp.full((16,),
                                scalar_at(iidx_v, b0, x) & (LANE_COLS - 1),
                                jnp.int32)
                glo = plsc.load_gather(ubufs.at[s], [lane, ucol])
                ghi = plsc.load_gather(ubufs.at[s], [lane + 16, ucol])
                vlo = plsc.load_gather(ibufs.at[s], [lane, icol])
                vhi = plsc.load_gather(ibufs.at[s], [lane + 16, icol])
                pu = glo * vlo + ghi * vhi
                r = lax.reduce_sum(pu, axes=(0,))

                y = x + NB
                ybase, yoff = (b0, y) if y < 16 else (b0 + 16, y - 16)

                @pl.when(b + NB < B_PER_W)
                def _():
                    fire(scalar_at(uidx_v, ybase, yoff),
                         scalar_at(iidx_v, ybase, yoff), s)

                accv = jnp.where(lane == x, r, accv)
        plsc.store_scatter(out_v, [b0 + lane], accv)
        return 0

    lax.fori_loop(0, B_PER_W // 16, g_body, 0)
    pltpu.sync_copy(out_v, out_hbm.at[pl.ds(base, B_PER_W)])


@jax.jit
def kernel(user_indices, item_indices, w_user, w_item):
    user_indices = user_indices.astype(jnp.int32)
    item_indices = item_indices.astype(jnp.int32)
    mesh = plsc.VectorSubcoreMesh(core_axis_name="c", subcore_axis_name="s")
    run = pl.kernel(
        _pmf_body,
        out_type=jax.ShapeDtypeStruct((BATCH,), jnp.float32),
        mesh=mesh,
        compiler_params=pltpu.CompilerParams(needs_layout_passes=False,
                                             use_tc_tiling_on_sc=True),
        scratch_types=[
            pltpu.VMEM((B_PER_W,), jnp.int32),
            pltpu.VMEM((B_PER_W,), jnp.int32),
            pltpu.VMEM((NB, LATENT_DIM, PAD_COLS), jnp.float32),
            pltpu.VMEM((NB, LATENT_DIM, PAD_COLS), jnp.float32),
            pltpu.VMEM((B_PER_W,), jnp.float32),
            pltpu.SemaphoreType.DMA((NB,)),
            pltpu.SemaphoreType.DMA((NB,)),
        ],
    )
    return run(user_indices, item_indices, w_user.T, w_item.T)


# speedup vs baseline: 1.0292x; 1.0292x over previous
import jax
import jax.numpy as jnp
from jax import lax
from jax.experimental import pallas as pl
from jax.experimental.pallas import tpu as pltpu
from jax.experimental.pallas import tpu_sc as plsc

LATENT_DIM = 32
BATCH = 16384
NUM_CORES = 2
NUM_SUBCORES = 16
NUM_WORKERS = NUM_CORES * NUM_SUBCORES
B_PER_W = BATCH // NUM_WORKERS
NB = 8
LANE_COLS = 128
PAD_COLS = 128


def _pmf_body(uidx_hbm, iidx_hbm, wut_hbm, wit_hbm, out_hbm,
              uidx_v, iidx_v, ubufs, ibufs, out_v, usems, isems):
    wid = lax.axis_index("s") * NUM_CORES + lax.axis_index("c")
    base = wid * B_PER_W

    pltpu.sync_copy(uidx_hbm.at[pl.ds(base, B_PER_W)], uidx_v)
    pltpu.sync_copy(iidx_hbm.at[pl.ds(base, B_PER_W)], iidx_v)

    lane = lax.iota(jnp.int32, 16)

    def scalar_at(ref, chunk_base, off):
        return ref[pl.ds(chunk_base, 16)][off]

    def fire(iu, ii, slot):
        ucol0 = pl.multiple_of((iu >> 7) * LANE_COLS, LANE_COLS)
        icol0 = pl.multiple_of((ii >> 7) * LANE_COLS, LANE_COLS)
        pltpu.async_copy(wut_hbm.at[:, pl.ds(ucol0, LANE_COLS)],
                         ubufs.at[slot, :, pl.ds(0, LANE_COLS)],
                         usems.at[slot])
        pltpu.async_copy(wit_hbm.at[:, pl.ds(icol0, LANE_COLS)],
                         ibufs.at[slot, :, pl.ds(0, LANE_COLS)],
                         isems.at[slot])

    for s in range(NB):
        fire(scalar_at(uidx_v, 0, s), scalar_at(iidx_v, 0, s), s)

    def g_body(g, _):
        b0 = g * 16
        accv = jnp.zeros((16,), jnp.float32)
        for q in range(16 // NB):
            for s in range(NB):
                b = b0 + q * NB + s
                pltpu.make_async_copy(
                    wut_hbm.at[:, pl.ds(0, LANE_COLS)],
                    ubufs.at[s, :, pl.ds(0, LANE_COLS)], usems.at[s]).wait()
                pltpu.make_async_copy(
                    wit_hbm.at[:, pl.ds(0, LANE_COLS)],
                    ibufs.at[s, :, pl.ds(0, LANE_COLS)], isems.at[s]).wait()
                x = q * NB + s
                ucol = jnp.full((16,),
                                scalar_at(uidx_v, b0, x) & (LANE_COLS - 1),
                                jnp.int32)
                icol = jnp.full((16,),
                                scalar_at(iidx_v, b0, x) & (LANE_COLS - 1),
                                jnp.int32)
                glo = plsc.load_gather(ubufs.at[s], [lane, ucol])
                ghi = plsc.load_gather(ubufs.at[s], [lane + 16, ucol])
                vlo = plsc.load_gather(ibufs.at[s], [lane, icol])
                vhi = plsc.load_gather(ibufs.at[s], [lane + 16, icol])
                pu = glo * vlo + ghi * vhi
                r = lax.reduce_sum(pu, axes=(0,))

                y = x + NB
                ybase, yoff = (b0, y) if y < 16 else (b0 + 16, y - 16)

                @pl.when(b + NB < B_PER_W)
                def _():
                    fire(scalar_at(uidx_v, ybase, yoff),
                         scalar_at(iidx_v, ybase, yoff), s)

                accv = jnp.where(lane == x, r, accv)
        plsc.store_scatter(out_v, [b0 + lane], accv)
        return 0

    lax.fori_loop(0, B_PER_W // 16, g_body, 0)
    pltpu.sync_copy(out_v, out_hbm.at[pl.ds(base, B_PER_W)])


@jax.jit
def kernel(user_indices, item_indices, w_user, w_item):
    user_indices = user_indices.astype(jnp.int32)
    item_indices = item_indices.astype(jnp.int32)
    mesh = plsc.VectorSubcoreMesh(core_axis_name="c", subcore_axis_name="s")
    run = pl.kernel(
        _pmf_body,
        out_type=jax.ShapeDtypeStruct((BATCH,), jnp.float32),
        mesh=mesh,
        compiler_params=pltpu.CompilerParams(needs_layout_passes=False,
                                             use_tc_tiling_on_sc=True),
        scratch_types=[
            pltpu.VMEM((B_PER_W,), jnp.int32),
            pltpu.VMEM((B_PER_W,), jnp.int32),
            pltpu.VMEM((NB, LATENT_DIM, PAD_COLS), jnp.float32),
            pltpu.VMEM((NB, LATENT_DIM, PAD_COLS), jnp.float32),
            pltpu.VMEM((B_PER_W,), jnp.float32),
            pltpu.SemaphoreType.DMA((NB,)),
            pltpu.SemaphoreType.DMA((NB,)),
        ],
    )
    return run(user_indices, item_indices, w_user.T, w_item.T)
